# baseline (speedup 1.0000x reference)
import jax
import jax.numpy as jnp
from jax import lax
from jax.experimental import pallas as pl
from jax.experimental.pallas import tpu as pltpu
import jax.experimental.pallas.tpu_sc as plsc

B = 1024
H = 128
D = 256
NC = 2
NS = 16
NW = NC * NS
IPW = B // NW
LANES = 16
CH = 64
NBUF = 3
HPC = H // CH
NUNITS = IPW * HPC


def _body(x_hbm, enc_hbm, dec_hbm, out_hbm, x_v, enc_t, dec_t, idx_v, bufs, ssem):
    w = lax.axis_index("s") * NC + lax.axis_index("c")
    base = pl.multiple_of(w * IPW, IPW)

    pltpu.sync_copy(x_hbm.at[pl.ds(base, IPW)], x_v)
    pltpu.sync_copy(enc_hbm, enc_t)
    pltpu.sync_copy(dec_hbm, dec_t)

    xv = [x_v[pl.ds(i * LANES, LANES)] for i in range(IPW // LANES)]
    for item in range(IPW):
        xid = xv[item // LANES][item % LANES]
        for c in range(H // LANES):
            v = enc_t[xid, pl.ds(c * LANES, LANES)]
            iv = jnp.clip(v.astype(jnp.int32), 0, H - 1)
            idx_v[item, pl.ds(c * LANES, LANES)] = iv

    def outer(o, carry):
        for b in range(NBUF):
            g = o * NBUF + b
            item = g // HPC
            half = g % HPC

            @pl.when(o > 0)
            def _wait_prev():
                pltpu.make_async_copy(
                    bufs.at[b], out_hbm.at[0, pl.ds(0, CH)], ssem.at[b]
                ).wait()

            def row_group(jg, c2):
                rvec = idx_v[item, pl.ds(half * CH + jg * LANES, LANES)]
                for l in range(LANES):
                    rid = rvec[l]
                    j = jg * LANES + l
                    vals = [
                        dec_t[rid, pl.ds(k * LANES, LANES)]
                        for k in range(D // LANES)
                    ]
                    for k in range(D // LANES):
                        bufs[b, j, pl.ds(k * LANES, LANES)] = vals[k]
                return c2

            lax.fori_loop(0, CH // LANES, row_group, 0)
            pltpu.async_copy(
                bufs.at[b],
                out_hbm.at[base + item, pl.ds(half * CH, CH)],
                ssem.at[b],
            )
        return carry

    lax.fori_loop(0, NUNITS // NBUF, outer, 0)
    for b in range(NBUF):
        pltpu.make_async_copy(
            bufs.at[b], out_hbm.at[0, pl.ds(0, CH)], ssem.at[b]
        ).wait()


def kernel(x, enc_table, dec_table):
    mesh = plsc.VectorSubcoreMesh(
        core_axis_name="c", subcore_axis_name="s", num_cores=NC, num_subcores=NS
    )
    run = pl.kernel(
        _body,
        out_type=jax.ShapeDtypeStruct((B, H, D), jnp.float32),
        mesh=mesh,
        compiler_params=pltpu.CompilerParams(disable_bounds_checks=True),
        scratch_types=[
            pltpu.VMEM((IPW,), jnp.int32),
            pltpu.VMEM((H * 2, H), jnp.float32),
            pltpu.VMEM((H, D), jnp.float32),
            pltpu.VMEM((IPW, H), jnp.int32),
            pltpu.VMEM((NBUF, CH, D), jnp.float32),
            pltpu.SemaphoreType.DMA((NBUF,)),
        ],
    )
    return run(x, enc_table, dec_table)


# speedup vs baseline: 1.1555x; 1.1555x over previous
import jax
import jax.numpy as jnp
from jax import lax
from jax.experimental import pallas as pl
from jax.experimental.pallas import tpu as pltpu
import jax.experimental.pallas.tpu_sc as plsc

B = 1024
H = 128
D = 256
NC = 2
NS = 16
NW = NC * NS
IPW = B // NW
LANES = 16
CH = 64
NBUF = 2
HPC = H // CH
NUNITS = IPW * HPC


def _body(x_hbm, enc_hbm, dec_hbm, out_hbm, x_v, enc_t, dec_t, idx_v, bufs, ssem):
    w = lax.axis_index("s") * NC + lax.axis_index("c")
    base = pl.multiple_of(w * IPW, IPW)

    pltpu.sync_copy(x_hbm.at[pl.ds(base, IPW)], x_v)
    pltpu.sync_copy(enc_hbm, enc_t)
    pltpu.sync_copy(dec_hbm, dec_t)

    xv = [x_v[pl.ds(i * LANES, LANES)] for i in range(IPW // LANES)]
    for item in range(IPW):
        xid = xv[item // LANES][item % LANES]
        for c in range(H // LANES):
            v = enc_t[xid, pl.ds(c * LANES, LANES)]
            iv = jnp.clip(v.astype(jnp.int32), 0, H - 1)
            idx_v[item, pl.ds(c * LANES, LANES)] = iv

    def outer(o, carry):
        for b in range(NBUF):
            g = o * NBUF + b
            item = g // HPC
            half = g % HPC

            @pl.when(o > 0)
            def _wait_prev():
                pltpu.make_async_copy(
                    bufs.at[b], out_hbm.at[0, pl.ds(0, CH)], ssem.at[b]
                ).wait()

            def row_group(jg, c2):
                rvec = idx_v[item, pl.ds(half * CH + jg * LANES, LANES)]
                for l in range(LANES):
                    rid = rvec[l]
                    j = jg * LANES + l
                    vals = [
                        dec_t[rid, pl.ds(k * LANES, LANES)]
                        for k in range(D // LANES)
                    ]
                    for k in range(D // LANES):
                        bufs[b, j, pl.ds(k * LANES, LANES)] = vals[k]
                return c2

            lax.fori_loop(0, CH // LANES, row_group, 0)
            pltpu.async_copy(
                bufs.at[b],
                out_hbm.at[base + item, pl.ds(half * CH, CH)],
                ssem.at[b],
            )
        return carry

    lax.fori_loop(0, NUNITS // NBUF, outer, 0)
    for b in range(NBUF):
        pltpu.make_async_copy(
            bufs.at[b], out_hbm.at[0, pl.ds(0, CH)], ssem.at[b]
        ).wait()


def kernel(x, enc_table, dec_table):
    mesh = plsc.VectorSubcoreMesh(
        core_axis_name="c", subcore_axis_name="s", num_cores=NC, num_subcores=NS
    )
    run = pl.kernel(
        _body,
        out_type=jax.ShapeDtypeStruct((B, H, D), jnp.float32),
        mesh=mesh,
        compiler_params=pltpu.CompilerParams(disable_bounds_checks=True),
        scratch_types=[
            pltpu.VMEM((IPW,), jnp.int32),
            pltpu.VMEM((H * 2, H), jnp.float32),
            pltpu.VMEM((H, D), jnp.float32),
            pltpu.VMEM((IPW, H), jnp.int32),
            pltpu.VMEM((NBUF, CH, D), jnp.float32),
            pltpu.SemaphoreType.DMA((NBUF,)),
        ],
    )
    return run(x, enc_table, dec_table)
